# baseline (speedup 1.0000x reference)
import numpy as np

import jax
import jax.numpy as jnp
from jax import lax
from jax.experimental import pallas as pl
from jax.experimental.pallas import tpu as pltpu
from jax.experimental.pallas import tpu_sc as plsc

N_NODES = 10000
N_EDGES = 320000
D_FEAT = 128
DH = D_FEAT // 2
NT = 16
C = 128
NI = 157
EP = NT * NI * C
NG = (NI - 3) // 2
RPT = 624

_SIGMA = np.array(list(range(0, 32, 2)) + list(range(1, 32, 2)) +
                  list(range(32, 64, 2)) + list(range(33, 64, 2)))
_PBLK = np.empty(64, np.int32)
_PBLK[_SIGMA] = np.arange(64, dtype=np.int32)
_PERM = np.concatenate([_PBLK, _PBLK + 64])

_SIG32 = np.array(list(range(0, 32, 2)) + list(range(1, 32, 2)))
_P32 = np.empty(32, np.int32)
_P32[_SIG32] = np.arange(32, dtype=np.int32)

_GATHER_DNUMS = lax.GatherDimensionNumbers(
    offset_dims=(), collapsed_slice_dims=(0,), start_index_map=(0,))


def _lane_bcast(vec, lane):
    idx = jnp.full((16, 1), lane, jnp.int32)
    return lax.gather(vec, idx, _GATHER_DNUMS, slice_sizes=(1,),
                      mode=lax.GatherScatterMode.PROMISE_IN_BOUNDS)


def _body(xA, xB, src3, dst3, w3, out, acc, xs, srcb, dstb, wb,
          dc0, dc1, g0, g1, f0, f1, gs0, gs1, ss0, ss1):
    c = lax.axis_index("c")
    s = lax.axis_index("s")
    r0 = s * RPT
    dstc = (dc0, dc1)
    gbuf = (g0, g1)
    fbuf = (f0, f1)
    gsem = (gs0, gs1)
    ssem = (ss0, ss1)

    def drain_g(b):
        pltpu.make_async_copy(xA.at[pl.ds(0, C)], gbuf[b], gsem[b]).wait()

    def drain_s(b):
        pltpu.make_async_copy(fbuf[b], acc.at[pl.ds(0, C)], ssem[b]).wait()

    def gather(i, b):
        pltpu.async_copy(xs.at[srcb.at[i]], gbuf[b], gsem[b])

    def scatter(i, b):
        for j in range(C // 32):
            d32 = dstb[i, pl.ds(j * 32, 32)]
            da, db = plsc.unpack(d32, format=plsc.PackFormat.INTERLEAVED,
                                 preferred_element_type=jnp.int32)
            dstc[b][pl.ds(j * 32, 16)] = da
            dstc[b][pl.ds(j * 32 + 16, 16)] = db
        pltpu.async_copy(fbuf[b], acc.at[dstc[b]], ssem[b], add=True)

    def compute(i, b):
        gb, fb = gbuf[b], fbuf[b]
        for j in range(C // 32):
            w32 = wb[i, pl.ds(j * 32, 32)]
            wa, wo = plsc.unpack(w32, format=plsc.PackFormat.INTERLEAVED,
                                 preferred_element_type=jnp.float32)
            for m in range(32):
                r = j * 32 + m
                wk = _lane_bcast(wa if m % 2 == 0 else wo, m // 2)
                for h in range(DH // 32):
                    v = gb[r, pl.ds(h * 32, 32)]
                    a, bb = plsc.unpack(v, format=plsc.PackFormat.INTERLEAVED,
                                        preferred_element_type=jnp.float32)
                    fb[r, pl.ds(h * 32, 16)] = a * wk
                    fb[r, pl.ds(h * 32 + 16, 16)] = bb * wk

    def chunk_step(i, b, first, traced):
        drain_g(b)
        if not first:
            drain_s(b)
        compute(i, b)
        scatter(i, b)
        if traced:
            @pl.when(i + 2 < NI)
            def _g():
                gather(i + 2, b)
        elif i + 2 < NI:
            gather(i + 2, b)

    pltpu.sync_copy(src3.at[s], srcb)
    pltpu.sync_copy(dst3.at[s], dstb)
    pltpu.sync_copy(w3.at[s], wb)

    @pl.when(c == 0)
    def _sxA():
        pltpu.sync_copy(xA.at[pl.ds(s * RPT, RPT)], xs.at[pl.ds(s * RPT, RPT)])

    @pl.when(c == 1)
    def _sxB():
        pltpu.sync_copy(xB.at[pl.ds(s * RPT, RPT)], xs.at[pl.ds(s * RPT, RPT)])

    @pl.when((s == NT - 1) & (c == 0))
    def _sxremA():
        pltpu.sync_copy(xA.at[pl.ds(NT * RPT, N_NODES - NT * RPT)],
                        xs.at[pl.ds(NT * RPT, N_NODES - NT * RPT)])

    @pl.when((s == NT - 1) & (c == 1))
    def _sxremB():
        pltpu.sync_copy(xB.at[pl.ds(NT * RPT, N_NODES - NT * RPT)],
                        xs.at[pl.ds(NT * RPT, N_NODES - NT * RPT)])

    zero = jnp.zeros((16,), jnp.float32)

    def zrow(r, carry):
        for q in range(DH // 16):
            f0[r, pl.ds(q * 16, 16)] = zero
        return carry

    lax.fori_loop(0, C, zrow, None)
    for k in range(RPT // C):
        pltpu.sync_copy(f0.at[:], acc.at[pl.ds(r0 + k * C, C)])
    tail = RPT % C
    pltpu.sync_copy(f0.at[pl.ds(0, tail)],
                    acc.at[pl.ds(r0 + (RPT // C) * C, tail)])
    rem = N_NODES - NT * RPT

    @pl.when(s == NT - 1)
    def _zero_rem():
        pltpu.sync_copy(f0.at[pl.ds(0, rem)],
                        acc.at[pl.ds(NT * RPT, rem)])

    plsc.subcore_barrier()

    gather(0, 0)
    gather(1, 1)
    chunk_step(0, 0, True, False)
    chunk_step(1, 1, True, False)

    def group(g, carry):
        i0 = 2 * g + 2
        chunk_step(i0, 0, False, True)
        chunk_step(i0 + 1, 1, False, True)
        return carry

    lax.fori_loop(0, NG, group, None)
    chunk_step(NI - 1, (NI - 1) % 2, False, False)
    drain_s((NI - 2) % 2)
    drain_s((NI - 1) % 2)

    plsc.subcore_barrier()

    pltpu.sync_copy(acc.at[pl.ds(r0, RPT)],
                    out.at[pl.ds(r0, RPT), pl.ds(c * DH, DH)])

    @pl.when(s == NT - 1)
    def _write_rem():
        pltpu.sync_copy(acc.at[pl.ds(NT * RPT, rem)],
                        out.at[pl.ds(NT * RPT, rem), pl.ds(c * DH, DH)])


_sc_spmm = pl.kernel(
    _body,
    out_type=jax.ShapeDtypeStruct((N_NODES, D_FEAT), jnp.float32),
    mesh=plsc.VectorSubcoreMesh(core_axis_name="c", subcore_axis_name="s"),
    scratch_types=(
        [pltpu.VMEM_SHARED((N_NODES, DH), jnp.float32)] +
        [pltpu.VMEM_SHARED((N_NODES, DH), jnp.bfloat16)] +
        [pltpu.VMEM((NI, C), jnp.int32)] +
        [pltpu.VMEM((NI, C), jnp.int16)] +
        [pltpu.VMEM((NI, C), jnp.bfloat16)] +
        [pltpu.VMEM((C,), jnp.int32)] * 2 +
        [pltpu.VMEM((C, DH), jnp.bfloat16)] * 2 +
        [pltpu.VMEM((C, DH), jnp.float32)] * 2 +
        [pltpu.SemaphoreType.DMA] * 4
    ),
    compiler_params=pltpu.CompilerParams(use_tc_tiling_on_sc=False,
                                         needs_layout_passes=False),
)


@jax.jit
def kernel(x, edge_index, edge_weight):
    pad = EP - N_EDGES
    s0 = jnp.pad(edge_index[0], (0, pad))
    d0 = jnp.pad(edge_index[1], (0, pad))
    w0 = jnp.pad(edge_weight, (0, pad))
    src = s0.reshape(NT, NI, C)
    dst = (d0.reshape(-1, 32)[:, _P32].astype(jnp.int16)
           .reshape(NT, NI, C))
    w = w0.astype(jnp.bfloat16).reshape(NT, NI, C)
    xp = x[:, _PERM].astype(jnp.bfloat16)
    xA = xp[:, :DH]
    xB = xp[:, DH:]
    return _sc_spmm(xA, xB, src, dst, w)
